# baseline (speedup 1.0000x reference)
import functools

import jax
import jax.numpy as jnp
from jax import lax
from jax.experimental import pallas as pl
from jax.experimental.pallas import tpu as pltpu
from jax.experimental.pallas import tpu_sc as plsc

BATCH = 16384
D = 64

_info = plsc.get_sparse_core_info()
NC, NS = _info.num_cores, _info.num_subcores
NW = NC * NS
B_PER_W = BATCH // NW


def _make_gather_kernel():
    mesh = plsc.VectorSubcoreMesh(core_axis_name="c", subcore_axis_name="s")

    @functools.partial(
        pl.kernel,
        mesh=mesh,
        out_type=(
            jax.ShapeDtypeStruct((BATCH, D), jnp.float32),
            jax.ShapeDtypeStruct((BATCH, D), jnp.float32),
        ),
        scratch_types=[
            pltpu.VMEM((B_PER_W,), jnp.int32),
            pltpu.VMEM((B_PER_W, D), jnp.float32),
            pltpu.VMEM((B_PER_W,), jnp.int32),
            pltpu.VMEM((B_PER_W, D), jnp.float32),
            pltpu.SemaphoreType.DMA,
            pltpu.SemaphoreType.DMA,
        ],
        compiler_params=pltpu.CompilerParams(use_tc_tiling_on_sc=False),
    )
    def gather_kernel(
        user_hbm,
        item_hbm,
        uw_hbm,
        iw_hbm,
        uout_hbm,
        iout_hbm,
        uidx_v,
        urows_v,
        iidx_v,
        irows_v,
        usem,
        isem,
    ):
        wid = lax.axis_index("s") * NC + lax.axis_index("c")
        base = wid * B_PER_W
        pltpu.sync_copy(user_hbm.at[pl.ds(base, B_PER_W)], uidx_v)
        pltpu.sync_copy(item_hbm.at[pl.ds(base, B_PER_W)], iidx_v)
        ucp = pltpu.async_copy(uw_hbm.at[uidx_v], urows_v, usem)
        icp = pltpu.async_copy(iw_hbm.at[iidx_v], irows_v, isem)
        ucp.wait()
        pltpu.sync_copy(urows_v, uout_hbm.at[pl.ds(base, B_PER_W)])
        icp.wait()
        pltpu.sync_copy(irows_v, iout_hbm.at[pl.ds(base, B_PER_W)])

    return gather_kernel


_gather = _make_gather_kernel()


@jax.jit
def kernel(user, item, user_weight, item_weight):
    user = user.astype(jnp.int32)
    item = item.astype(jnp.int32)
    return _gather(user, item, user_weight, item_weight)


# speedup vs baseline: 1.5822x; 1.5822x over previous
import functools

import jax
import jax.numpy as jnp
from jax import lax
from jax.experimental import pallas as pl
from jax.experimental.pallas import tpu as pltpu
from jax.experimental.pallas import tpu_sc as plsc

BATCH = 16384
D = 64
N_USERS = 1000001
N_ITEMS = 1000000

_info = plsc.get_sparse_core_info()
NC, NS = _info.num_cores, _info.num_subcores
NW = NC * NS
B_PER_W = BATCH // NW
CH = 256
N_CHUNKS = B_PER_W // CH


def _make_gather_kernel():
    mesh = plsc.VectorSubcoreMesh(core_axis_name="c", subcore_axis_name="s")

    @functools.partial(
        pl.kernel,
        mesh=mesh,
        out_type=(
            jax.ShapeDtypeStruct((BATCH, D), jnp.float32),
            jax.ShapeDtypeStruct((BATCH, D), jnp.float32),
        ),
        scratch_types=[
            pltpu.VMEM((B_PER_W,), jnp.int32),
            pltpu.VMEM((CH, D), jnp.float32),
            pltpu.VMEM((B_PER_W,), jnp.int32),
            pltpu.VMEM((CH, D), jnp.float32),
            pltpu.SemaphoreType.DMA,
            pltpu.SemaphoreType.DMA,
        ],
    )
    def gather_kernel(
        user_hbm,
        item_hbm,
        uw_hbm,
        iw_hbm,
        uout_hbm,
        iout_hbm,
        uidx_v,
        urows_v,
        iidx_v,
        irows_v,
        usem,
        isem,
    ):
        wid = lax.axis_index("s") * NC + lax.axis_index("c")
        base = wid * B_PER_W
        pltpu.sync_copy(user_hbm.at[pl.ds(base, B_PER_W)], uidx_v)
        pltpu.sync_copy(item_hbm.at[pl.ds(base, B_PER_W)], iidx_v)

        for c in range(N_CHUNKS):
            off = c * CH

            def issue_u(g, _):
                v = uidx_v[pl.ds(off + g * 16, 16)]
                for k in range(16):
                    pltpu.async_copy(
                        uw_hbm.at[pl.ds(v[k], 1), :],
                        urows_v.at[pl.ds(g * 16 + k, 1), :],
                        usem,
                    )
                return 0

            def issue_i(g, _):
                v = iidx_v[pl.ds(off + g * 16, 16)]
                for k in range(16):
                    pltpu.async_copy(
                        iw_hbm.at[pl.ds(v[k], 1), :],
                        irows_v.at[pl.ds(g * 16 + k, 1), :],
                        isem,
                    )
                return 0

            lax.fori_loop(0, CH // 16, issue_u, 0)
            lax.fori_loop(0, CH // 16, issue_i, 0)

            def drain_u(j, _):
                pltpu.make_async_copy(
                    uw_hbm.at[pl.ds(0, 1), :], urows_v.at[pl.ds(j, 1), :], usem
                ).wait()
                return 0

            def drain_i(j, _):
                pltpu.make_async_copy(
                    iw_hbm.at[pl.ds(0, 1), :], irows_v.at[pl.ds(j, 1), :], isem
                ).wait()
                return 0

            lax.fori_loop(0, CH, drain_u, 0)
            pltpu.sync_copy(urows_v, uout_hbm.at[pl.ds(base + off, CH)])
            lax.fori_loop(0, CH, drain_i, 0)
            pltpu.sync_copy(irows_v, iout_hbm.at[pl.ds(base + off, CH)])

    return gather_kernel


_gather = _make_gather_kernel()


@jax.jit
def kernel(user, item, user_weight, item_weight):
    user = user.astype(jnp.int32)
    item = item.astype(jnp.int32)
    return _gather(user, item, user_weight, item_weight)
